# Initial kernel scaffold; baseline (speedup 1.0000x reference)
import jax
import jax.numpy as jnp
from jax.experimental import pallas as pl


def kernel(emb, dates, emb_table):
    raise NotImplementedError("write your pallas kernel here")


# speedup vs baseline: 1.7973x; 1.7973x over previous
import jax
import jax.numpy as jnp
from jax import lax
from jax.experimental import pallas as pl
from jax.experimental.pallas import tpu as pltpu
from jax.experimental.pallas import tpu_sc as plsc

D = 64
CHUNK = 512
GSUB = 128
NW = 32


SUPER = 1024


def _pe_body(emb_hbm, dates_hbm, table_hbm, out_hbm, idx_v, pos_v, emb_v, sem):
    wid = lax.axis_index("s") * 2 + lax.axis_index("c")
    n_rows = emb_hbm.shape[0]
    rows_per_w = n_rows // NW
    base = wid * rows_per_w
    num_chunks = rows_per_w // SUPER

    def chunk_body(c, carry):
        rbase = base + c * SUPER
        ibase = pl.multiple_of(rbase // GSUB, 8)
        pltpu.sync_copy(dates_hbm.at[pl.ds(ibase, SUPER // GSUB)], idx_v)
        for h in range(SUPER // CHUNK):
            hbase = pl.multiple_of(rbase + h * CHUNK, 512)
            copies = []
            for j in range(CHUNK // GSUB):
                copies.append(
                    pltpu.async_copy(
                        table_hbm.at[idx_v.at[h * (CHUNK // GSUB) + j]],
                        pos_v.at[pl.ds(j * GSUB, GSUB)],
                        sem,
                    )
                )
            pltpu.sync_copy(emb_hbm.at[pl.ds(hbase, CHUNK)], emb_v)
            for cp in copies:
                cp.wait()

            def row_body(r, carry2):
                for g in range(D // 16):
                    sl = pl.ds(g * 16, 16)
                    emb_v[r, sl] = emb_v[r, sl] + pos_v[r, sl]
                return carry2

            lax.fori_loop(0, CHUNK, row_body, 0, unroll=2)
            pltpu.sync_copy(emb_v, out_hbm.at[pl.ds(hbase, CHUNK)])
        return carry

    lax.fori_loop(0, num_chunks, chunk_body, 0)


def kernel(emb, dates, emb_table):
    B, L, d = emb.shape
    N = B * L
    emb2 = emb.reshape(N, d)
    dates2 = dates.reshape(N // GSUB, GSUB)

    mesh = plsc.VectorSubcoreMesh(core_axis_name="c", subcore_axis_name="s")
    pe = pl.kernel(
        _pe_body,
        out_type=jax.ShapeDtypeStruct((N, d), jnp.float32),
        mesh=mesh,
        compiler_params=pltpu.CompilerParams(use_tc_tiling_on_sc=False),
        scratch_types=[
            pltpu.VMEM((SUPER // GSUB, GSUB), jnp.int32),
            pltpu.VMEM((CHUNK, D), jnp.float32),
            pltpu.VMEM((CHUNK, D), jnp.float32),
            pltpu.SemaphoreType.DMA,
        ],
    )
    out = pe(emb2, dates2, emb_table)
    return out.reshape(B, L, d)


# speedup vs baseline: 2.1929x; 1.2201x over previous
import jax
import jax.numpy as jnp
from jax import lax
from jax.experimental import pallas as pl
from jax.experimental.pallas import tpu as pltpu
from jax.experimental.pallas import tpu_sc as plsc

D = 64
CHUNK = 512
GSUB = 128
NW = 32


SUPER = 1024


def _pe_body(emb_hbm, dates_hbm, table_hbm, out_hbm, idx_v, emb_v, sem):
    wid = lax.axis_index("s") * 2 + lax.axis_index("c")
    n_rows = emb_hbm.shape[0]
    rows_per_w = n_rows // NW
    base = wid * rows_per_w
    num_chunks = rows_per_w // SUPER

    def chunk_body(c, carry):
        rbase = base + c * SUPER
        ibase = pl.multiple_of(rbase // GSUB, 8)
        pltpu.sync_copy(dates_hbm.at[pl.ds(ibase, SUPER // GSUB)], idx_v)
        for h in range(SUPER // CHUNK):
            hbase = pl.multiple_of(rbase + h * CHUNK, 512)
            pltpu.sync_copy(emb_hbm.at[pl.ds(hbase, CHUNK)], emb_v)
            copies = []
            for j in range(CHUNK // GSUB):
                copies.append(
                    pltpu.async_copy(
                        table_hbm.at[idx_v.at[h * (CHUNK // GSUB) + j]],
                        emb_v.at[pl.ds(j * GSUB, GSUB)],
                        sem,
                        add=True,
                    )
                )
            for cp in copies:
                cp.wait()
            pltpu.sync_copy(emb_v, out_hbm.at[pl.ds(hbase, CHUNK)])
        return carry

    lax.fori_loop(0, num_chunks, chunk_body, 0)


def kernel(emb, dates, emb_table):
    B, L, d = emb.shape
    N = B * L
    emb2 = emb.reshape(N, d)
    dates2 = dates.reshape(N // GSUB, GSUB)

    mesh = plsc.VectorSubcoreMesh(core_axis_name="c", subcore_axis_name="s")
    pe = pl.kernel(
        _pe_body,
        out_type=jax.ShapeDtypeStruct((N, d), jnp.float32),
        mesh=mesh,
        compiler_params=pltpu.CompilerParams(use_tc_tiling_on_sc=False),
        scratch_types=[
            pltpu.VMEM((SUPER // GSUB, GSUB), jnp.int32),
            pltpu.VMEM((CHUNK, D), jnp.float32),
            pltpu.SemaphoreType.DMA,
        ],
    )
    out = pe(emb2, dates2, emb_table)
    return out.reshape(B, L, d)


# speedup vs baseline: 2.1981x; 1.0024x over previous
import jax
import jax.numpy as jnp
from jax import lax
from jax.experimental import pallas as pl
from jax.experimental.pallas import tpu as pltpu
from jax.experimental.pallas import tpu_sc as plsc

D = 64
CHUNK = 512
GSUB = 128
SUPER = 1024
NW = 32
NGS = CHUNK // GSUB


def _pe_body(emb_hbm, dates_hbm, table_hbm, out_hbm,
             idx_v, emb_v, semi, seme, semg, semo):
    wid = lax.axis_index("s") * 2 + lax.axis_index("c")
    n_rows = emb_hbm.shape[0]
    rows_per_w = n_rows // NW
    base = wid * rows_per_w
    nblocks = rows_per_w // SUPER
    nhalf = SUPER // CHUNK

    def idx_copy(b):
        ibase = pl.multiple_of((base + b * SUPER) // GSUB, 8)
        return pltpu.make_async_copy(
            dates_hbm.at[pl.ds(ibase, SUPER // GSUB)],
            idx_v.at[lax.rem(b, 2)], semi)

    def emb_copy(k):
        hbase = pl.multiple_of(base + k * CHUNK, CHUNK)
        return pltpu.make_async_copy(
            emb_hbm.at[pl.ds(hbase, CHUNK)],
            emb_v.at[lax.rem(k, 2)], seme)

    def out_copy(k):
        hbase = pl.multiple_of(base + k * CHUNK, CHUNK)
        return pltpu.make_async_copy(
            emb_v.at[lax.rem(k, 2)],
            out_hbm.at[pl.ds(hbase, CHUNK)], semo)

    idx_copy(0).start()
    emb_copy(0).start()

    def block_body(b, carry):
        sb = lax.rem(b, 2)

        @pl.when(b < nblocks - 1)
        def _():
            idx_copy(b + 1).start()

        for h in range(nhalf):
            k = b * nhalf + h
            s = lax.rem(k, 2)
            if h == 0:
                idx_copy(b).wait()
            emb_copy(k).wait()
            gathers = []
            for j in range(NGS):
                gathers.append(
                    pltpu.async_copy(
                        table_hbm.at[idx_v.at[sb, h * NGS + j]],
                        emb_v.at[s, pl.ds(j * GSUB, GSUB)],
                        semg,
                        add=True,
                    )
                )
            if h == 0:
                @pl.when(b >= 1)
                def _():
                    out_copy(k - 1).wait()
                emb_copy(k + 1).start()
            else:
                out_copy(k - 1).wait()

                @pl.when(b < nblocks - 1)
                def _():
                    emb_copy(k + 1).start()
            for g in gathers:
                g.wait()
            out_copy(k).start()
        return carry

    lax.fori_loop(0, nblocks, block_body, 0)
    out_copy(nblocks * nhalf - 1).wait()


def kernel(emb, dates, emb_table):
    B, L, d = emb.shape
    N = B * L
    emb2 = emb.reshape(N, d)
    dates2 = dates.reshape(N // GSUB, GSUB)

    mesh = plsc.VectorSubcoreMesh(core_axis_name="c", subcore_axis_name="s")
    pe = pl.kernel(
        _pe_body,
        out_type=jax.ShapeDtypeStruct((N, d), jnp.float32),
        mesh=mesh,
        compiler_params=pltpu.CompilerParams(use_tc_tiling_on_sc=False),
        scratch_types=[
            pltpu.VMEM((2, SUPER // GSUB, GSUB), jnp.int32),
            pltpu.VMEM((2, CHUNK, D), jnp.float32),
            pltpu.SemaphoreType.DMA,
            pltpu.SemaphoreType.DMA,
            pltpu.SemaphoreType.DMA,
            pltpu.SemaphoreType.DMA,
        ],
    )
    out = pe(emb2, dates2, emb_table)
    return out.reshape(B, L, d)


# speedup vs baseline: 3.1333x; 1.4254x over previous
import jax
import jax.numpy as jnp
from jax import lax
from jax.experimental import pallas as pl
from jax.experimental.pallas import tpu as pltpu
from jax.experimental.pallas import tpu_sc as plsc

D = 64
CHUNK = 128
GSUB = 128
SUPER = 1024
NW = 32
HPB = SUPER // CHUNK


def _pe_body(emb_hbm, dates_hbm, table_hbm, out_hbm,
             idx_v, emb_v, pos_v, semi, seme, semg, semo):
    wid = lax.axis_index("s") * 2 + lax.axis_index("c")
    n_rows = emb_hbm.shape[0]
    rows_per_w = n_rows // NW
    base = wid * rows_per_w
    nblocks = rows_per_w // SUPER
    nchunks = rows_per_w // CHUNK

    def idx_copy(b):
        ibase = pl.multiple_of((base + b * SUPER) // GSUB, 8)
        return pltpu.make_async_copy(
            dates_hbm.at[pl.ds(ibase, SUPER // GSUB)],
            idx_v.at[lax.rem(b, 2)], semi)

    def emb_copy(k):
        hbase = pl.multiple_of(base + k * CHUNK, CHUNK)
        return pltpu.make_async_copy(
            emb_hbm.at[pl.ds(hbase, CHUNK)],
            emb_v.at[lax.rem(k, 2)], seme)

    def gather_copy(k):
        b, h = k // HPB, lax.rem(k, HPB)
        return pltpu.make_async_copy(
            table_hbm.at[idx_v.at[lax.rem(b, 2), h]],
            pos_v.at[lax.rem(k, 2)], semg)

    def out_copy(k):
        hbase = pl.multiple_of(base + k * CHUNK, CHUNK)
        return pltpu.make_async_copy(
            emb_v.at[lax.rem(k, 2)],
            out_hbm.at[pl.ds(hbase, CHUNK)], semo)

    idx_copy(0).start()
    idx_copy(1).start()
    idx_copy(0).wait()
    emb_copy(0).start()
    gather_copy(0).start()

    def chunk_body(k, carry):
        s = lax.rem(k, 2)

        @pl.when((lax.rem(k, HPB) == HPB - 1) & (k // HPB + 2 < nblocks))
        def _():
            idx_copy(k // HPB + 2).start()

        @pl.when(k >= 1)
        def _():
            out_copy(k - 1).wait()

        @pl.when(k + 1 < nchunks)
        def _():
            emb_copy(k + 1).start()

            @pl.when(lax.rem(k + 1, HPB) == 0)
            def _():
                idx_copy(k // HPB + 1).wait()

            gather_copy(k + 1).start()

        emb_copy(k).wait()
        gather_copy(k).wait()

        def row_body(r, carry2):
            for g in range(D // 16):
                sl = pl.ds(g * 16, 16)
                emb_v[s, r, sl] = emb_v[s, r, sl] + pos_v[s, r, sl]
            return carry2

        lax.fori_loop(0, CHUNK, row_body, 0, unroll=2)
        out_copy(k).start()
        return carry

    lax.fori_loop(0, nchunks, chunk_body, 0)
    out_copy(nchunks - 1).wait()


def kernel(emb, dates, emb_table):
    B, L, d = emb.shape
    N = B * L
    emb2 = emb.reshape(N, d)
    dates2 = dates.reshape(N // GSUB, GSUB)
    table128 = jnp.pad(emb_table, ((0, 0), (0, 128 - d)))

    mesh = plsc.VectorSubcoreMesh(core_axis_name="c", subcore_axis_name="s")
    pe = pl.kernel(
        _pe_body,
        out_type=jax.ShapeDtypeStruct((N, d), jnp.float32),
        mesh=mesh,
        scratch_types=[
            pltpu.VMEM((2, SUPER // GSUB, GSUB), jnp.int32),
            pltpu.VMEM((2, CHUNK, D), jnp.float32),
            pltpu.VMEM((2, CHUNK, 128), jnp.float32),
            pltpu.SemaphoreType.DMA,
            pltpu.SemaphoreType.DMA,
            pltpu.SemaphoreType.DMA,
            pltpu.SemaphoreType.DMA,
        ],
    )
    out = pe(emb2, dates2, table128)
    return out.reshape(B, L, d)


# speedup vs baseline: 3.4802x; 1.1107x over previous
import jax
import jax.numpy as jnp
from jax import lax
from jax.experimental import pallas as pl
from jax.experimental.pallas import tpu as pltpu
from jax.experimental.pallas import tpu_sc as plsc

D = 64
BW = 128
LCH = 4
LBLK = 8
NW = 32


def _pe_body(emb_hbm, dates_hbm, table_hbm, out_hbm,
             table_v, idx_v, emb_v, semi, seme, semo):
    wid = lax.axis_index("s") * 2 + lax.axis_index("c")
    L = emb_hbm.shape[0]
    wb = pl.multiple_of(wid * BW, BW)
    nchunks = L // LCH
    nblocks = L // LBLK

    def idx_copy(blk):
        lbase = pl.multiple_of(blk * LBLK, LBLK)
        return pltpu.make_async_copy(
            dates_hbm.at[pl.ds(lbase, LBLK), pl.ds(wb, BW)],
            idx_v.at[lax.rem(blk, 2)], semi)

    def emb_copy(k):
        lbase = pl.multiple_of(k * LCH, LCH)
        return pltpu.make_async_copy(
            emb_hbm.at[pl.ds(lbase, LCH), :, pl.ds(wb, BW)],
            emb_v.at[lax.rem(k, 2)], seme)

    def out_copy(k):
        lbase = pl.multiple_of(k * LCH, LCH)
        return pltpu.make_async_copy(
            emb_v.at[lax.rem(k, 2)],
            out_hbm.at[pl.ds(lbase, LCH), :, pl.ds(wb, BW)], semo)

    idx_copy(0).start()
    idx_copy(1).start()
    emb_copy(0).start()
    pltpu.sync_copy(table_hbm, table_v)
    idx_copy(0).wait()

    def chunk_body(k, carry):
        s = lax.rem(k, 2)
        blk = k // 2
        sb = lax.rem(blk, 2)

        @pl.when(k >= 1)
        def _():
            out_copy(k - 1).wait()

        @pl.when(k + 1 < nchunks)
        def _():
            emb_copy(k + 1).start()

            @pl.when(lax.rem(k + 1, 2) == 0)
            def _():
                idx_copy(blk + 1).wait()

        emb_copy(k).wait()

        for li in range(LCH):
            lrow = s * LCH + li
            for g in range(BW // 16):
                sl = pl.ds(g * 16, 16)
                idx16 = idx_v[sb, lrow, sl]

                def dbody(dd, carry2):
                    dsplat = jnp.full((16,), dd, dtype=jnp.int32)
                    v = plsc.load_gather(table_v, [dsplat, idx16])
                    emb_v[s, li, dd, sl] = emb_v[s, li, dd, sl] + v
                    return carry2

                lax.fori_loop(0, D, dbody, 0, unroll=8)

        @pl.when((lax.rem(k, 2) == 1) & (blk + 2 < nblocks))
        def _():
            idx_copy(blk + 2).start()

        out_copy(k).start()
        return carry

    lax.fori_loop(0, nchunks, chunk_body, 0)
    out_copy(nchunks - 1).wait()


def kernel(emb, dates, emb_table):
    B, L, d = emb.shape
    emb_t = jnp.transpose(emb, (1, 2, 0))
    dates_t = dates.T
    table_t = emb_table.T

    mesh = plsc.VectorSubcoreMesh(core_axis_name="c", subcore_axis_name="s")
    pe = pl.kernel(
        _pe_body,
        out_type=jax.ShapeDtypeStruct((L, d, B), jnp.float32),
        mesh=mesh,
        compiler_params=pltpu.CompilerParams(needs_layout_passes=False),
        scratch_types=[
            pltpu.VMEM((d, emb_table.shape[0]), jnp.float32),
            pltpu.VMEM((2, LBLK, BW), jnp.int32),
            pltpu.VMEM((2, LCH, D, BW), jnp.float32),
            pltpu.SemaphoreType.DMA,
            pltpu.SemaphoreType.DMA,
            pltpu.SemaphoreType.DMA,
        ],
    )
    out_t = pe(emb_t, dates_t, table_t)
    return jnp.transpose(out_t, (2, 0, 1))


# speedup vs baseline: 3.8257x; 1.0993x over previous
import jax
import jax.numpy as jnp
from jax import lax
from jax.experimental import pallas as pl
from jax.experimental.pallas import tpu as pltpu
from jax.experimental.pallas import tpu_sc as plsc

D = 64
BW = 128
LBLK = 8
NW = 32
VPAD = 512


def _pe_body(emb_hbm, dates_hbm, table_hbm, out_hbm,
             table_v, idx_v, emb_v, semt, semi, seme, semo):
    wid = lax.axis_index("s") * 2 + lax.axis_index("c")
    L = emb_hbm.shape[0]
    wb = pl.multiple_of(wid * BW, BW)
    nblocks = L // LBLK

    def idx_copy(blk):
        lbase = pl.multiple_of(blk * LBLK, LBLK)
        return pltpu.make_async_copy(
            dates_hbm.at[pl.ds(lbase, LBLK), pl.ds(wb, BW)],
            idx_v.at[lax.rem(blk, 2)], semi)

    def emb_copy(k):
        return pltpu.make_async_copy(
            emb_hbm.at[k, :, pl.ds(wb, BW)],
            emb_v.at[lax.rem(k, 2)], seme)

    def out_copy(k):
        return pltpu.make_async_copy(
            emb_v.at[lax.rem(k, 2)],
            out_hbm.at[k, :, pl.ds(wb, BW)], semo)

    tcopies = [
        pltpu.make_async_copy(
            table_hbm.at[dd], table_v.at[pl.ds(dd * VPAD, VPAD)], semt)
        for dd in range(D)
    ]
    for cp in tcopies:
        cp.start()
    idx_copy(0).start()
    idx_copy(1).start()
    emb_copy(0).start()
    for cp in tcopies:
        cp.wait()
    idx_copy(0).wait()

    def chunk_body(k, carry):
        s = lax.rem(k, 2)
        blk = k // LBLK
        lrow = lax.rem(k, LBLK)

        @pl.when(k >= 1)
        def _():
            out_copy(k - 1).wait()

        @pl.when(k + 1 < L)
        def _():
            emb_copy(k + 1).start()

            @pl.when(lax.rem(k + 1, LBLK) == 0)
            def _():
                idx_copy(blk + 1).wait()

        emb_copy(k).wait()

        for g in range(BW // 16):
            sl = pl.ds(g * 16, 16)
            idx16 = idx_v[lax.rem(blk, 2), lrow, sl]
            for dd in range(D):
                v = plsc.load_gather(table_v, [idx16 + dd * VPAD])
                plsc.addupdate(emb_v.at[s, dd, sl], v)

        @pl.when((lrow == LBLK - 1) & (blk + 2 < nblocks))
        def _():
            idx_copy(blk + 2).start()

        out_copy(k).start()
        return carry

    lax.fori_loop(0, L, chunk_body, 0)
    out_copy(L - 1).wait()


def kernel(emb, dates, emb_table):
    B, L, d = emb.shape
    emb_t = jnp.transpose(emb, (1, 2, 0))
    dates_t = dates.T
    table_t = jnp.pad(emb_table.T, ((0, 0), (0, VPAD - emb_table.shape[0])))

    mesh = plsc.VectorSubcoreMesh(core_axis_name="c", subcore_axis_name="s")
    pe = pl.kernel(
        _pe_body,
        out_type=jax.ShapeDtypeStruct((L, d, B), jnp.float32),
        mesh=mesh,
        compiler_params=pltpu.CompilerParams(needs_layout_passes=False),
        scratch_types=[
            pltpu.VMEM((D * VPAD,), jnp.float32),
            pltpu.VMEM((2, LBLK, BW), jnp.int32),
            pltpu.VMEM((2, D, BW), jnp.float32),
            pltpu.SemaphoreType.DMA,
            pltpu.SemaphoreType.DMA,
            pltpu.SemaphoreType.DMA,
            pltpu.SemaphoreType.DMA,
        ],
    )
    out_t = pe(emb_t, dates_t, table_t)
    return jnp.transpose(out_t, (2, 0, 1))


# speedup vs baseline: 12.0589x; 3.1521x over previous
import jax
import jax.numpy as jnp
from jax import lax
from jax.experimental import pallas as pl
from jax.experimental.pallas import tpu as pltpu
from jax.experimental.pallas import tpu_sc as plsc

D = 64
BW = 128
LBLK = 8
NW = 32
VPAD = 512


def _pe_body(emb_hbm, dates_hbm, table_hbm, out_hbm,
             table_v, idx_v, emb_v, semt, semi, seme, semo):
    wid = lax.axis_index("s") * 2 + lax.axis_index("c")
    L = emb_hbm.shape[0]
    wb = pl.multiple_of(wid * BW, BW)
    nblocks = L // LBLK

    def idx_copy(blk):
        lbase = pl.multiple_of(blk * LBLK, LBLK)
        return pltpu.make_async_copy(
            dates_hbm.at[pl.ds(lbase, LBLK), pl.ds(wb, BW)],
            idx_v.at[lax.rem(blk, 2)], semi)

    def emb_copy(k):
        return pltpu.make_async_copy(
            emb_hbm.at[k, :, pl.ds(wb, BW)],
            emb_v.at[lax.rem(k, 2)], seme)

    def out_copy(k):
        return pltpu.make_async_copy(
            emb_v.at[lax.rem(k, 2)],
            out_hbm.at[k, :, pl.ds(wb, BW)], semo)

    tcopies = [
        pltpu.make_async_copy(
            table_hbm.at[dd], table_v.at[pl.ds(dd * VPAD, VPAD)], semt)
        for dd in range(D)
    ]
    for cp in tcopies:
        cp.start()
    idx_copy(0).start()
    idx_copy(1).start()
    emb_copy(0).start()
    for cp in tcopies:
        cp.wait()
    idx_copy(0).wait()

    def chunk_body(k, carry):
        s = lax.rem(k, 2)
        blk = k // LBLK
        lrow = lax.rem(k, LBLK)

        @pl.when(k >= 1)
        def _():
            out_copy(k - 1).wait()

        @pl.when(k + 1 < L)
        def _():
            emb_copy(k + 1).start()

            @pl.when(lax.rem(k + 1, LBLK) == 0)
            def _():
                idx_copy(blk + 1).wait()

        emb_copy(k).wait()

        NG = BW // 16
        sls = [pl.ds(g * 16, 16) for g in range(NG)]
        idx16s = [idx_v[lax.rem(blk, 2), lrow, sls[g]] for g in range(NG)]
        prev_vs = None
        for dd in range(D):
            vs = [plsc.load_gather(table_v, [idx16s[g] + dd * VPAD])
                  for g in range(NG)]
            if prev_vs is not None:
                for g in range(NG):
                    plsc.addupdate(emb_v.at[s, dd - 1, sls[g]], prev_vs[g])
            prev_vs = vs
        for g in range(NG):
            plsc.addupdate(emb_v.at[s, D - 1, sls[g]], prev_vs[g])

        @pl.when((lrow == LBLK - 1) & (blk + 2 < nblocks))
        def _():
            idx_copy(blk + 2).start()

        out_copy(k).start()
        return carry

    lax.fori_loop(0, L, chunk_body, 0)
    out_copy(L - 1).wait()


def kernel(emb, dates, emb_table):
    B, L, d = emb.shape
    emb_t = jnp.transpose(emb, (1, 2, 0))
    dates_t = dates.T
    table_t = jnp.pad(emb_table.T, ((0, 0), (0, VPAD - emb_table.shape[0])))

    mesh = plsc.VectorSubcoreMesh(core_axis_name="c", subcore_axis_name="s")
    pe = pl.kernel(
        _pe_body,
        out_type=jax.ShapeDtypeStruct((L, d, B), jnp.float32),
        mesh=mesh,
        compiler_params=pltpu.CompilerParams(needs_layout_passes=False),
        scratch_types=[
            pltpu.VMEM((D * VPAD,), jnp.float32),
            pltpu.VMEM((2, LBLK, BW), jnp.int32),
            pltpu.VMEM((2, D, BW), jnp.float32),
            pltpu.SemaphoreType.DMA,
            pltpu.SemaphoreType.DMA,
            pltpu.SemaphoreType.DMA,
            pltpu.SemaphoreType.DMA,
        ],
    )
    out_t = pe(emb_t, dates_t, table_t)
    return jnp.transpose(out_t, (2, 0, 1))


# speedup vs baseline: 16.9641x; 1.4068x over previous
import jax
import jax.numpy as jnp
from jax import lax
from jax.experimental import pallas as pl
from jax.experimental.pallas import tpu as pltpu
from jax.experimental.pallas import tpu_sc as plsc

D = 64
BW = 128
LBLK = 8
NW = 32
VPAD = 512


def _pe_body(emb_hbm, dates_hbm, table_hbm, out_hbm,
             table_v, idx_v, emb_v, semt, semi, seme, semo):
    wid = lax.axis_index("s") * 2 + lax.axis_index("c")
    L = emb_hbm.shape[0]
    wb = pl.multiple_of(wid * BW, BW)
    nblocks = L // LBLK

    def idx_copy(blk):
        lbase = pl.multiple_of(blk * LBLK, LBLK)
        return pltpu.make_async_copy(
            dates_hbm.at[pl.ds(lbase, LBLK), pl.ds(wb, BW)],
            idx_v.at[lax.rem(blk, 2)], semi)

    def emb_copy(k):
        return pltpu.make_async_copy(
            emb_hbm.at[k, :, pl.ds(wb, BW)],
            emb_v.at[lax.rem(k, 4)], seme)

    def out_copy(k):
        return pltpu.make_async_copy(
            emb_v.at[lax.rem(k, 4)],
            out_hbm.at[k, :, pl.ds(wb, BW)], semo)

    tcopies = [
        pltpu.make_async_copy(
            table_hbm.at[dd], table_v.at[pl.ds(dd * VPAD, VPAD)], semt)
        for dd in range(D)
    ]
    for cp in tcopies:
        cp.start()
    idx_copy(0).start()
    idx_copy(1).start()
    emb_copy(0).start()
    emb_copy(1).start()
    for cp in tcopies:
        cp.wait()
    idx_copy(0).wait()

    def chunk_body(k, carry):
        s = lax.rem(k, 4)
        blk = k // LBLK
        lrow = lax.rem(k, LBLK)

        @pl.when(k >= 2)
        def _():
            out_copy(k - 2).wait()

        @pl.when(k + 2 < L)
        def _():
            emb_copy(k + 2).start()

        @pl.when((k + 1 < L) & (lax.rem(k + 1, LBLK) == 0))
        def _():
            idx_copy(blk + 1).wait()

        emb_copy(k).wait()

        NG = BW // 16
        sls = [pl.ds(g * 16, 16) for g in range(NG)]
        idx16s = [idx_v[lax.rem(blk, 2), lrow, sls[g]] for g in range(NG)]
        prev_vs = None
        for dd in range(D):
            vs = [plsc.load_gather(table_v, [idx16s[g] + dd * VPAD])
                  for g in range(NG)]
            if prev_vs is not None:
                for g in range(NG):
                    plsc.addupdate(emb_v.at[s, dd - 1, sls[g]], prev_vs[g])
            prev_vs = vs
        for g in range(NG):
            plsc.addupdate(emb_v.at[s, D - 1, sls[g]], prev_vs[g])

        @pl.when((lrow == LBLK - 1) & (blk + 2 < nblocks))
        def _():
            idx_copy(blk + 2).start()

        out_copy(k).start()
        return carry

    lax.fori_loop(0, L, chunk_body, 0)
    out_copy(L - 2).wait()
    out_copy(L - 1).wait()


def kernel(emb, dates, emb_table):
    B, L, d = emb.shape
    emb_t = jnp.transpose(emb, (1, 2, 0))
    dates_t = dates.T
    table_t = jnp.pad(emb_table.T, ((0, 0), (0, VPAD - emb_table.shape[0])))

    mesh = plsc.VectorSubcoreMesh(core_axis_name="c", subcore_axis_name="s")
    pe = pl.kernel(
        _pe_body,
        out_type=jax.ShapeDtypeStruct((L, d, B), jnp.float32),
        mesh=mesh,
        compiler_params=pltpu.CompilerParams(needs_layout_passes=False),
        scratch_types=[
            pltpu.VMEM((D * VPAD,), jnp.float32),
            pltpu.VMEM((2, LBLK, BW), jnp.int32),
            pltpu.VMEM((4, D, BW), jnp.float32),
            pltpu.SemaphoreType.DMA,
            pltpu.SemaphoreType.DMA,
            pltpu.SemaphoreType.DMA,
            pltpu.SemaphoreType.DMA,
        ],
    )
    out_t = pe(emb_t, dates_t, table_t)
    return jnp.transpose(out_t, (2, 0, 1))
